# baseline (speedup 1.0000x reference)
import functools

import jax
import jax.numpy as jnp
from jax import lax
from jax.experimental import pallas as pl
from jax.experimental.pallas import tpu as pltpu
from jax.experimental.pallas import tpu_sc as plsc

N_NODES = 10000
D_IN = 128
D_HID = 128
N_CLASSES = 40
DCP = 48

NC = 2
NS = 16
NW = NC * NS
CH = 128
NSTR = 1
CH2 = CH // NSTR
SPLIT0 = 0.6
NBUF = 2
N_PAD = 10112
RPT = N_PAD // NS
DUMMY = N_NODES


def _row_chunks(total, step=CH):
    out, off = [], 0
    while off < total:
        c = min(step, total - off)
        out.append((off, c))
        off += c
    return out


def _mesh():
    return plsc.VectorSubcoreMesh(
        core_axis_name="c", subcore_axis_name="s",
        num_cores=NC, num_subcores=NS)


def _sc_degrees(src_t, dst_t, k0, k1):
    K = src_t.shape[1]

    @functools.partial(
        pl.kernel,
        out_type=jax.ShapeDtypeStruct((NC, 2, N_PAD, 16), jnp.float32),
        mesh=_mesh(),
        compiler_params=pltpu.CompilerParams(use_tc_tiling_on_sc=False),
        scratch_types=(
            pltpu.VMEM((K, NSTR, CH2), jnp.int32),
            pltpu.VMEM((K, NSTR, CH2), jnp.int32),
            pltpu.VMEM((CH2, 16), jnp.float32),
            pltpu.VMEM((CH2, 16), jnp.float32),
            pltpu.VMEM_SHARED((N_PAD, 16), jnp.float32),
            pltpu.VMEM_SHARED((N_PAD, 16), jnp.float32),
            pltpu.SemaphoreType.DMA,
            pltpu.SemaphoreType.DMA,
        ),
    )
    def deg_kernel(src_hbm, dst_hbm, out_hbm,
                   idx_s, idx_d, ones, zeros, acc_o, acc_i, sem_o, sem_i):
        cid = lax.axis_index("c")
        sid = lax.axis_index("s")
        t = cid * NS + sid
        pltpu.sync_copy(src_hbm.at[t], idx_s)
        pltpu.sync_copy(dst_hbm.at[t], idx_d)

        def fill(r, _):
            ones[r, :] = jnp.ones((16,), jnp.float32)
            zeros[r, :] = jnp.zeros((16,), jnp.float32)
            return 0
        lax.fori_loop(0, CH2, fill, 0)

        base = sid * RPT
        for off, c in _row_chunks(RPT, CH2):
            pltpu.sync_copy(zeros.at[pl.ds(0, c)],
                            acc_o.at[pl.ds(base + off, c)])
            pltpu.sync_copy(zeros.at[pl.ds(0, c)],
                            acc_i.at[pl.ds(base + off, c)])
        plsc.subcore_barrier()

        def deg_pipe(kc):
            def step(k, _):
                @pl.when(k >= 2)
                def _():
                    for h in range(NSTR):
                        pltpu.make_async_copy(
                            ones, acc_o.at[idx_s.at[k - 2, h]], sem_o).wait()
                        pltpu.make_async_copy(
                            ones, acc_i.at[idx_d.at[k - 2, h]], sem_i).wait()
                for h in range(NSTR):
                    pltpu.async_copy(
                        ones, acc_o.at[idx_s.at[k, h]], sem_o, add=True)
                    pltpu.async_copy(
                        ones, acc_i.at[idx_d.at[k, h]], sem_i, add=True)
                return 0
            lax.fori_loop(0, kc, step, 0)

            def drain(k, _):
                for h in range(NSTR):
                    pltpu.make_async_copy(
                        ones, acc_o.at[idx_s.at[k, h]], sem_o).wait()
                    pltpu.make_async_copy(
                        ones, acc_i.at[idx_d.at[k, h]], sem_i).wait()
                return 0
            lax.fori_loop(max(kc - 2, 0), kc, drain, 0)

        @pl.when(cid == 0)
        def _():
            deg_pipe(k0)

        @pl.when(cid == 1)
        def _():
            deg_pipe(k1)
        plsc.subcore_barrier()

        for off, c in _row_chunks(RPT):
            pltpu.sync_copy(acc_o.at[pl.ds(base + off, c)],
                            out_hbm.at[cid, 0, pl.ds(base + off, c)])
            pltpu.sync_copy(acc_i.at[pl.ds(base + off, c)],
                            out_hbm.at[cid, 1, pl.ds(base + off, c)])

    return deg_kernel(src_t, dst_t)


def _sc_msgpass2(tabs, widths, src_t, dst_t, k0, k1):
    K = src_t.shape[1]
    nst = len(widths)

    @functools.partial(
        pl.kernel,
        out_type=tuple(jax.ShapeDtypeStruct((NC, N_PAD, w), jnp.float32)
                       for w in widths),
        mesh=_mesh(),
        compiler_params=pltpu.CompilerParams(use_tc_tiling_on_sc=False),
        scratch_types=(
            pltpu.VMEM((K, NSTR, CH2), jnp.int32),
            pltpu.VMEM((K, NSTR, CH2), jnp.int32),
        ) + tuple(pltpu.VMEM((NBUF, CH2, w), jnp.float32) for w in widths)
          + tuple(pltpu.VMEM((CH2, w), jnp.float32) for w in widths)
          + tuple(pltpu.VMEM_SHARED((N_PAD, w), jnp.float32) for w in widths)
          + (pltpu.SemaphoreType.DMA,) * (4 * nst),
    )
    def mp_kernel(*args):
        tabs_hbm = args[:nst]
        src_hbm, dst_hbm = args[nst], args[nst + 1]
        outs_hbm = args[nst + 2:2 * nst + 2]
        idx_s, idx_d = args[2 * nst + 2], args[2 * nst + 3]
        rows = args[2 * nst + 4:3 * nst + 4]
        zbufs = args[3 * nst + 4:4 * nst + 4]
        accs = args[4 * nst + 4:5 * nst + 4]
        sems = args[5 * nst + 4:]
        cid = lax.axis_index("c")
        sid = lax.axis_index("s")
        t = cid * NS + sid
        pltpu.sync_copy(src_hbm.at[t], idx_s)
        pltpu.sync_copy(dst_hbm.at[t], idx_d)

        def zrow(r, _):
            for z, w in zip(zbufs, widths):
                for j in range(w // 16):
                    z[r, pl.ds(j * 16, 16)] = jnp.zeros((16,), jnp.float32)
            return 0
        lax.fori_loop(0, CH2, zrow, 0)

        base = sid * RPT
        for off, c in _row_chunks(RPT, CH2):
            for z, acc in zip(zbufs, accs):
                pltpu.sync_copy(z.at[pl.ds(0, c)],
                                acc.at[pl.ds(base + off, c)])
        plsc.subcore_barrier()

        def gfire(k, b):
            for i in range(nst):
                pltpu.async_copy(tabs_hbm[i].at[idx_s.at[k, 0]],
                                 rows[i].at[b], sems[4 * i + b])

        def gwait(k, b):
            for i in range(nst):
                pltpu.make_async_copy(tabs_hbm[i].at[idx_s.at[k, 0]],
                                      rows[i].at[b], sems[4 * i + b]).wait()

        def sfire(k, b):
            for i in range(nst):
                pltpu.async_copy(rows[i].at[b], accs[i].at[idx_d.at[k, 0]],
                                 sems[4 * i + 2 + b], add=True)

        def swait(k, b):
            for i in range(nst):
                pltpu.make_async_copy(rows[i].at[b],
                                      accs[i].at[idx_d.at[k, 0]],
                                      sems[4 * i + 2 + b]).wait()

        def pipe(kc):
            gfire(0, 0)

            def step(i, _):
                ka = 2 * i
                gfire(ka + 1, 1)
                gwait(ka, 0)
                sfire(ka, 0)
                swait(ka, 0)

                @pl.when(ka + 2 < kc)
                def _():
                    gfire(ka + 2, 0)
                gwait(ka + 1, 1)
                sfire(ka + 1, 1)
                swait(ka + 1, 1)
                return 0
            lax.fori_loop(0, kc // 2, step, 0)

        @pl.when(cid == 0)
        def _():
            pipe(k0)

        @pl.when(cid == 1)
        def _():
            pipe(k1)
        plsc.subcore_barrier()

        for off, c in _row_chunks(RPT):
            for acc, out in zip(accs, outs_hbm):
                pltpu.sync_copy(acc.at[pl.ds(base + off, c)],
                                out.at[cid, pl.ds(base + off, c)])

    return mp_kernel(*tabs, src_t, dst_t)


def _edge_layout(idx, k0, k1):
    n0 = NS * k0 * CH
    n1 = NS * k1 * CH
    pad = n0 + n1 - idx.shape[0]
    idxp = jnp.concatenate(
        [idx, jnp.full((pad,), DUMMY, jnp.int32)])
    kmax = max(k0, k1)

    def blk(a, k):
        a = a.reshape(NS, k, NSTR, CH2)
        if k < kmax:
            a = jnp.concatenate(
                [a, jnp.full((NS, kmax - k, NSTR, CH2), DUMMY, jnp.int32)],
                axis=1)
        return a
    return jnp.concatenate([blk(idxp[:n0], k0), blk(idxp[n0:], k1)], axis=0)


def _tc_pre(x_pad, degp):
    def body(x_ref, degp_ref, *outs):
        do = degp_ref[0, 0] + degp_ref[1, 0]
        s_out = lax.rsqrt(jnp.maximum(do[:, 0:1], 1.0))
        h = x_ref[...] * s_out
        for j, o in enumerate(outs):
            o[...] = h[:, 16 * j:16 * (j + 1)]
    nb = 8
    rb = N_PAD // nb
    return pl.pallas_call(
        body,
        grid=(nb,),
        in_specs=[pl.BlockSpec((rb, D_IN), lambda i: (i, 0)),
                  pl.BlockSpec((NC, 2, rb, 16), lambda i: (0, 0, i, 0))],
        out_specs=(pl.BlockSpec((rb, 16), lambda i: (i, 0)),) * 8,
        out_shape=(jax.ShapeDtypeStruct((N_PAD, 16), jnp.float32),) * 8,
    )(x_pad, degp)


def _tc_mid(aggs, degp, W0, b0r, W1p):
    def body(*refs):
        a = refs[:8]
        degp_ref, w0_ref, b0_ref, w1_ref = refs[8:12]
        outs = refs[12:]
        agg = jnp.concatenate([p[0] + p[1] for p in a], axis=1)
        di = degp_ref[0, 1] + degp_ref[1, 1]
        do = degp_ref[0, 0] + degp_ref[1, 0]
        s_in = lax.rsqrt(jnp.maximum(di[:, 0:1], 1.0))
        s_out = lax.rsqrt(jnp.maximum(do[:, 0:1], 1.0))
        h1 = jnp.dot(agg * s_in, w0_ref[...],
                     preferred_element_type=jnp.float32) + b0_ref[...]
        h1 = jnp.maximum(h1, 0.0)
        h2 = jnp.dot(h1 * s_out, w1_ref[...],
                     preferred_element_type=jnp.float32)
        for j, o in enumerate(outs):
            o[...] = h2[:, 16 * j:16 * (j + 1)]
    nb = 8
    rb = N_PAD // nb
    aspec = pl.BlockSpec((NC, rb, 16), lambda i: (0, i, 0))
    ospec = pl.BlockSpec((rb, 16), lambda i: (i, 0))
    return pl.pallas_call(
        body,
        grid=(nb,),
        in_specs=[aspec] * 8 + [
            pl.BlockSpec((NC, 2, rb, 16), lambda i: (0, 0, i, 0)),
            pl.BlockSpec((D_HID, D_HID), lambda i: (0, 0)),
            pl.BlockSpec((1, D_HID), lambda i: (0, 0)),
            pl.BlockSpec((D_HID, DCP), lambda i: (0, 0))],
        out_specs=(ospec,) * 3,
        out_shape=(jax.ShapeDtypeStruct((N_PAD, 16), jnp.float32),) * 3,
    )(*aggs, degp, W0, b0r, W1p)


def _tc_post(agg1s, degp, b1r):
    def body(a0, a1, a2, degp_ref, b1_ref, o_ref):
        di = degp_ref[0, 1] + degp_ref[1, 1]
        s_in = lax.rsqrt(jnp.maximum(di[:, 0:1], 1.0))
        agg = jnp.concatenate(
            [a0[0] + a0[1], a1[0] + a1[1], a2[0] + a2[1]], axis=1)
        o_ref[...] = agg * s_in + b1_ref[...]
    nb = 8
    rb = N_PAD // nb
    aspec = pl.BlockSpec((NC, rb, 16), lambda i: (0, i, 0))
    return pl.pallas_call(
        body,
        grid=(nb,),
        in_specs=[aspec, aspec, aspec,
                  pl.BlockSpec((NC, 2, rb, 16), lambda i: (0, 0, i, 0)),
                  pl.BlockSpec((1, DCP), lambda i: (0, 0))],
        out_specs=pl.BlockSpec((rb, DCP), lambda i: (i, 0)),
        out_shape=jax.ShapeDtypeStruct((N_PAD, DCP), jnp.float32),
    )(*agg1s, degp, b1r)


def kernel(inputs, edge_index, W0, b0, W1, b1):
    x = inputs
    e = edge_index.shape[1]
    ktot = -(-e // (NS * CH))
    k0 = max(NBUF, int(ktot * SPLIT0) // NBUF * NBUF)
    k1 = -(-(ktot - k0) // NBUF) * NBUF
    src_t = _edge_layout(edge_index[0], k0, k1)
    dst_t = _edge_layout(edge_index[1], k0, k1)

    degp = _sc_degrees(src_t, dst_t, k0, k1)

    x_pad = jnp.pad(x, ((0, N_PAD - x.shape[0]), (0, 0)))
    h0q = _tc_pre(x_pad, degp)
    agg0a = _sc_msgpass2(h0q[:4], (16,) * 4, src_t, dst_t, k0, k1)
    agg0b = _sc_msgpass2(h0q[4:], (16,) * 4, src_t, dst_t, k0, k1)

    W1p = jnp.pad(W1, ((0, 0), (0, DCP - N_CLASSES)))
    h2s = _tc_mid(agg0a + agg0b, degp, W0, b0.reshape(1, -1), W1p)
    agg1s = _sc_msgpass2(h2s, (16,) * 3, src_t, dst_t, k0, k1)

    b1p = jnp.pad(b1, (0, DCP - N_CLASSES)).reshape(1, -1)
    outp = _tc_post(agg1s, degp, b1p)
    return outp[:N_NODES, :N_CLASSES]


# speedup vs baseline: 1.1563x; 1.1563x over previous
import functools

import jax
import jax.numpy as jnp
from jax import lax
from jax.experimental import pallas as pl
from jax.experimental.pallas import tpu as pltpu
from jax.experimental.pallas import tpu_sc as plsc

N_NODES = 10000
D_IN = 128
D_HID = 128
N_CLASSES = 40
DCP = 48

NC = 2
NS = 16
NW = NC * NS
CH = 128
NSTR = 1
CH2 = CH // NSTR
SPLIT0 = 0.6
NBUF = 2
N_PAD = 10112
RPT = N_PAD // NS
DUMMY = N_NODES


def _row_chunks(total, step=CH):
    out, off = [], 0
    while off < total:
        c = min(step, total - off)
        out.append((off, c))
        off += c
    return out


def _mesh():
    return plsc.VectorSubcoreMesh(
        core_axis_name="c", subcore_axis_name="s",
        num_cores=NC, num_subcores=NS)


def _sc_degrees(src_t, dst_t, k0, k1):
    K = src_t.shape[1]

    @functools.partial(
        pl.kernel,
        out_type=jax.ShapeDtypeStruct((NC, 2, N_PAD, 16), jnp.float32),
        mesh=_mesh(),
        compiler_params=pltpu.CompilerParams(use_tc_tiling_on_sc=False),
        scratch_types=(
            pltpu.VMEM((K, NSTR, CH2), jnp.int32),
            pltpu.VMEM((K, NSTR, CH2), jnp.int32),
            pltpu.VMEM((CH2, 16), jnp.float32),
            pltpu.VMEM((CH2, 16), jnp.float32),
            pltpu.VMEM_SHARED((N_PAD, 16), jnp.float32),
            pltpu.VMEM_SHARED((N_PAD, 16), jnp.float32),
            pltpu.SemaphoreType.DMA,
            pltpu.SemaphoreType.DMA,
        ),
    )
    def deg_kernel(src_hbm, dst_hbm, out_hbm,
                   idx_s, idx_d, ones, zeros, acc_o, acc_i, sem_o, sem_i):
        cid = lax.axis_index("c")
        sid = lax.axis_index("s")
        t = cid * NS + sid
        pltpu.sync_copy(src_hbm.at[t], idx_s)
        pltpu.sync_copy(dst_hbm.at[t], idx_d)

        def fill(r, _):
            ones[r, :] = jnp.ones((16,), jnp.float32)
            zeros[r, :] = jnp.zeros((16,), jnp.float32)
            return 0
        lax.fori_loop(0, CH2, fill, 0)

        base = sid * RPT
        for off, c in _row_chunks(RPT, CH2):
            pltpu.sync_copy(zeros.at[pl.ds(0, c)],
                            acc_o.at[pl.ds(base + off, c)])
            pltpu.sync_copy(zeros.at[pl.ds(0, c)],
                            acc_i.at[pl.ds(base + off, c)])
        plsc.subcore_barrier()

        def deg_pipe(kc):
            def step(k, _):
                @pl.when(k >= 2)
                def _():
                    for h in range(NSTR):
                        pltpu.make_async_copy(
                            ones, acc_o.at[idx_s.at[k - 2, h]], sem_o).wait()
                        pltpu.make_async_copy(
                            ones, acc_i.at[idx_d.at[k - 2, h]], sem_i).wait()
                for h in range(NSTR):
                    pltpu.async_copy(
                        ones, acc_o.at[idx_s.at[k, h]], sem_o, add=True)
                    pltpu.async_copy(
                        ones, acc_i.at[idx_d.at[k, h]], sem_i, add=True)
                return 0
            lax.fori_loop(0, kc, step, 0)

            def drain(k, _):
                for h in range(NSTR):
                    pltpu.make_async_copy(
                        ones, acc_o.at[idx_s.at[k, h]], sem_o).wait()
                    pltpu.make_async_copy(
                        ones, acc_i.at[idx_d.at[k, h]], sem_i).wait()
                return 0
            lax.fori_loop(max(kc - 2, 0), kc, drain, 0)

        @pl.when(cid == 0)
        def _():
            deg_pipe(k0)

        @pl.when(cid == 1)
        def _():
            deg_pipe(k1)
        plsc.subcore_barrier()

        for off, c in _row_chunks(RPT):
            pltpu.sync_copy(acc_o.at[pl.ds(base + off, c)],
                            out_hbm.at[cid, 0, pl.ds(base + off, c)])
            pltpu.sync_copy(acc_i.at[pl.ds(base + off, c)],
                            out_hbm.at[cid, 1, pl.ds(base + off, c)])

    return deg_kernel(src_t, dst_t)


def _sc_msgpass2(tab_a, tab_b, src_t, dst_t, wa, wb, k0, k1):
    K = src_t.shape[1]

    @functools.partial(
        pl.kernel,
        out_type=(jax.ShapeDtypeStruct((NC, N_PAD, wa), jnp.float32),
                  jax.ShapeDtypeStruct((NC, N_PAD, wb), jnp.float32)),
        mesh=_mesh(),
        compiler_params=pltpu.CompilerParams(use_tc_tiling_on_sc=False),
        scratch_types=(
            pltpu.VMEM((K, NSTR, CH2), jnp.int32),
            pltpu.VMEM((K, NSTR, CH2), jnp.int32),
            pltpu.VMEM((NBUF, CH2, wa), jnp.float32),
            pltpu.VMEM((NBUF, CH2, wb), jnp.float32),
            pltpu.VMEM((CH2, wa), jnp.float32),
            pltpu.VMEM((CH2, wb), jnp.float32),
            pltpu.VMEM_SHARED((N_PAD, wa), jnp.float32),
            pltpu.VMEM_SHARED((N_PAD, wb), jnp.float32),
        ) + (pltpu.SemaphoreType.DMA,) * 8,
    )
    def mp_kernel(taba_hbm, tabb_hbm, src_hbm, dst_hbm, outa_hbm, outb_hbm,
                  idx_s, idx_d, rows_a, rows_b, zbuf_a, zbuf_b,
                  acc_a, acc_b, *sems):
        cid = lax.axis_index("c")
        sid = lax.axis_index("s")
        t = cid * NS + sid
        pltpu.sync_copy(src_hbm.at[t], idx_s)
        pltpu.sync_copy(dst_hbm.at[t], idx_d)

        def zrow(r, _):
            for j in range(wa // 16):
                zbuf_a[r, pl.ds(j * 16, 16)] = jnp.zeros((16,), jnp.float32)
            for j in range(wb // 16):
                zbuf_b[r, pl.ds(j * 16, 16)] = jnp.zeros((16,), jnp.float32)
            return 0
        lax.fori_loop(0, CH2, zrow, 0)

        base = sid * RPT
        for off, c in _row_chunks(RPT, CH2):
            pltpu.sync_copy(zbuf_a.at[pl.ds(0, c)],
                            acc_a.at[pl.ds(base + off, c)])
            pltpu.sync_copy(zbuf_b.at[pl.ds(0, c)],
                            acc_b.at[pl.ds(base + off, c)])
        plsc.subcore_barrier()

        streams = ((taba_hbm, rows_a, acc_a, 0), (tabb_hbm, rows_b, acc_b, 4))

        def gfire(k, b):
            for tab, rows, _, s in streams:
                pltpu.async_copy(tab.at[idx_s.at[k, 0]],
                                 rows.at[b], sems[s + b])

        def gwait(k, b):
            for tab, rows, _, s in streams:
                pltpu.make_async_copy(tab.at[idx_s.at[k, 0]],
                                      rows.at[b], sems[s + b]).wait()

        def sfire(k, b):
            for _, rows, acc, s in streams:
                pltpu.async_copy(rows.at[b], acc.at[idx_d.at[k, 0]],
                                 sems[s + 2 + b], add=True)

        def swait(k, b):
            for _, rows, acc, s in streams:
                pltpu.make_async_copy(rows.at[b], acc.at[idx_d.at[k, 0]],
                                      sems[s + 2 + b]).wait()

        def pipe(kc):
            gfire(0, 0)

            def step(i, _):
                ka = 2 * i
                gfire(ka + 1, 1)
                gwait(ka, 0)
                sfire(ka, 0)
                swait(ka, 0)

                @pl.when(ka + 2 < kc)
                def _():
                    gfire(ka + 2, 0)
                gwait(ka + 1, 1)
                sfire(ka + 1, 1)
                swait(ka + 1, 1)
                return 0
            lax.fori_loop(0, kc // 2, step, 0)

        @pl.when(cid == 0)
        def _():
            pipe(k0)

        @pl.when(cid == 1)
        def _():
            pipe(k1)
        plsc.subcore_barrier()

        for off, c in _row_chunks(RPT):
            pltpu.sync_copy(acc_a.at[pl.ds(base + off, c)],
                            outa_hbm.at[cid, pl.ds(base + off, c)])
            pltpu.sync_copy(acc_b.at[pl.ds(base + off, c)],
                            outb_hbm.at[cid, pl.ds(base + off, c)])

    return mp_kernel(tab_a, tab_b, src_t, dst_t)


def _edge_layout(idx, k0, k1):
    n0 = NS * k0 * CH
    n1 = NS * k1 * CH
    pad = n0 + n1 - idx.shape[0]
    idxp = jnp.concatenate(
        [idx, jnp.full((pad,), DUMMY, jnp.int32)])
    kmax = max(k0, k1)

    def blk(a, k):
        a = a.reshape(NS, k, NSTR, CH2)
        if k < kmax:
            a = jnp.concatenate(
                [a, jnp.full((NS, kmax - k, NSTR, CH2), DUMMY, jnp.int32)],
                axis=1)
        return a
    return jnp.concatenate([blk(idxp[:n0], k0), blk(idxp[n0:], k1)], axis=0)


def _tc_pre(x_pad, degp):
    def body(x_ref, degp_ref, o0, o1, o2, o3):
        do = degp_ref[0, 0] + degp_ref[1, 0]
        s_out = lax.rsqrt(jnp.maximum(do[:, 0:1], 1.0))
        h = x_ref[...] * s_out
        o0[...] = h[:, 0:32]
        o1[...] = h[:, 32:64]
        o2[...] = h[:, 64:96]
        o3[...] = h[:, 96:128]
    q = jax.ShapeDtypeStruct((N_PAD, 32), jnp.float32)
    return pl.pallas_call(body, out_shape=(q, q, q, q))(x_pad, degp)


def _tc_mid(aggs, degp, W0, b0r, W1p):
    def body(a0, a1, a2, a3, degp_ref, w0_ref, b0_ref, w1_ref, oa, ob):
        agg = jnp.concatenate(
            [a0[0] + a0[1], a1[0] + a1[1], a2[0] + a2[1], a3[0] + a3[1]],
            axis=1)
        di = degp_ref[0, 1] + degp_ref[1, 1]
        do = degp_ref[0, 0] + degp_ref[1, 0]
        s_in = lax.rsqrt(jnp.maximum(di[:, 0:1], 1.0))
        s_out = lax.rsqrt(jnp.maximum(do[:, 0:1], 1.0))
        h1 = jnp.dot(agg * s_in, w0_ref[...],
                     preferred_element_type=jnp.float32) + b0_ref[...]
        h1 = jnp.maximum(h1, 0.0)
        h2 = jnp.dot(h1 * s_out, w1_ref[...],
                     preferred_element_type=jnp.float32)
        oa[...] = h2[:, 0:32]
        ob[...] = h2[:, 32:DCP]
    nb = 8
    rb = N_PAD // nb
    aspec = pl.BlockSpec((NC, rb, 32), lambda i: (0, i, 0))
    return pl.pallas_call(
        body,
        grid=(nb,),
        in_specs=[aspec, aspec, aspec, aspec,
                  pl.BlockSpec((NC, 2, rb, 16), lambda i: (0, 0, i, 0)),
                  pl.BlockSpec((D_HID, D_HID), lambda i: (0, 0)),
                  pl.BlockSpec((1, D_HID), lambda i: (0, 0)),
                  pl.BlockSpec((D_HID, DCP), lambda i: (0, 0))],
        out_specs=(pl.BlockSpec((rb, 32), lambda i: (i, 0)),
                   pl.BlockSpec((rb, DCP - 32), lambda i: (i, 0))),
        out_shape=(jax.ShapeDtypeStruct((N_PAD, 32), jnp.float32),
                   jax.ShapeDtypeStruct((N_PAD, DCP - 32), jnp.float32)),
    )(*aggs, degp, W0, b0r, W1p)


def _tc_post(agg1a, agg1b, degp, b1r):
    def body(aa_ref, ab_ref, degp_ref, b1_ref, o_ref):
        di = degp_ref[0, 1] + degp_ref[1, 1]
        s_in = lax.rsqrt(jnp.maximum(di[:, 0:1], 1.0))
        agg = jnp.concatenate([aa_ref[0] + aa_ref[1], ab_ref[0] + ab_ref[1]],
                              axis=1)
        o_ref[...] = agg * s_in + b1_ref[...]
    return pl.pallas_call(
        body, out_shape=jax.ShapeDtypeStruct((N_PAD, DCP), jnp.float32),
    )(agg1a, agg1b, degp, b1r)


def kernel(inputs, edge_index, W0, b0, W1, b1):
    x = inputs
    e = edge_index.shape[1]
    ktot = -(-e // (NS * CH))
    k0 = max(NBUF, int(ktot * SPLIT0) // NBUF * NBUF)
    k1 = -(-(ktot - k0) // NBUF) * NBUF
    src_t = _edge_layout(edge_index[0], k0, k1)
    dst_t = _edge_layout(edge_index[1], k0, k1)

    degp = _sc_degrees(src_t, dst_t, k0, k1)

    x_pad = jnp.pad(x, ((0, N_PAD - x.shape[0]), (0, 0)))
    h0q = _tc_pre(x_pad, degp)
    agg00, agg01 = _sc_msgpass2(h0q[0], h0q[1], src_t, dst_t, 32, 32, k0, k1)
    agg02, agg03 = _sc_msgpass2(h0q[2], h0q[3], src_t, dst_t, 32, 32, k0, k1)

    W1p = jnp.pad(W1, ((0, 0), (0, DCP - N_CLASSES)))
    h2a, h2b = _tc_mid((agg00, agg01, agg02, agg03), degp,
                       W0, b0.reshape(1, -1), W1p)
    agg1a, agg1b = _sc_msgpass2(h2a, h2b, src_t, dst_t, 32, DCP - 32, k0, k1)

    b1p = jnp.pad(b1, (0, DCP - N_CLASSES)).reshape(1, -1)
    outp = _tc_post(agg1a, agg1b, degp, b1p)
    return outp[:N_NODES, :N_CLASSES]
